# baseline (speedup 1.0000x reference)
import jax
import jax.numpy as jnp
from jax import lax
from jax.experimental import pallas as pl
from jax.experimental.pallas import tpu as pltpu
from jax.experimental.pallas import tpu_sc as plsc

NC, NS, L = 2, 16, 16
NW = NC * NS


def _mesh():
    return plsc.VectorSubcoreMesh(
        core_axis_name="c", subcore_axis_name="s", num_cores=NC, num_subcores=NS
    )


def _params():
    return pltpu.CompilerParams(needs_layout_passes=False)


def _worker_id():
    return lax.axis_index("s") * NC + lax.axis_index("c")


def _build_posof(idx_v, posof_v, B):
    lid = lax.iota(jnp.int32, L)

    @pl.loop(0, B // L)
    def _(c):
        c_v = idx_v[pl.ds(c * L, L)]
        pos_v = c * L + lid
        for k in range(L):
            plsc.store_scatter(posof_v, [c_v], pos_v, mask=lid == k)


def _make_update(M, D, B):
    bpw = B // NW
    nch = bpw // L

    def body(val_hbm, idx_hbm, new_mem_ref, idx_v, posof_v, sidx, didx,
             rows, gsem, ssem):
        wid = _worker_id()
        base = wid * bpw

        pltpu.sync_copy(idx_hbm, idx_v)
        _build_posof(idx_v, posof_v, B)

        def stage(t):
            c_v = idx_v[pl.ds(base + t * L, L)]
            s_v = plsc.load_gather(posof_v, [c_v])
            p = t % 2
            sidx[p][...] = s_v
            didx[p][...] = c_v
            return pltpu.async_copy(val_hbm.at[sidx[p]], rows[p], gsem[p])

        def scatter(t):
            p = t % 2
            return pltpu.async_copy(
                rows[p], new_mem_ref.at[didx[p]], ssem[p]
            )

        gd = [None, None]
        sd = [None, None]
        gd[0] = stage(0)
        for t in range(nch):
            if t + 1 < nch:
                p2 = (t + 1) % 2
                if sd[p2] is not None:
                    sd[p2].wait()
                    sd[p2] = None
                gd[p2] = stage(t + 1)
            gd[t % 2].wait()
            sd[t % 2] = scatter(t)
        for d in sd:
            if d is not None:
                d.wait()

    return pl.kernel(
        body,
        out_type=(),
        mesh=_mesh(),
        scratch_types=[
            pltpu.VMEM((B,), jnp.int32),
            pltpu.VMEM((M,), jnp.int32),
            [pltpu.VMEM((L,), jnp.int32)] * 2,
            [pltpu.VMEM((L,), jnp.int32)] * 2,
            [pltpu.VMEM((L, D), jnp.float32)] * 2,
            [pltpu.SemaphoreType.DMA] * 2,
            [pltpu.SemaphoreType.DMA] * 2,
        ],
        compiler_params=_params(),
    )


def _make_retrieve(M, D, B, R):
    rpw = R // NW

    def body(mem_hbm, val_hbm, idx_hbm, ridx_hbm, out_hbm, idx_v, posof_v,
             ridx_v, rows_v, semg):
        wid = _worker_id()
        base = wid * rpw

        pltpu.sync_copy(idx_hbm, idx_v)
        pltpu.sync_copy(ridx_hbm.at[pl.ds(base, rpw)], ridx_v)

        neg1 = jnp.full((L,), -1, jnp.int32)

        @pl.loop(0, M // L)
        def _(i):
            posof_v[pl.ds(i * L, L)] = neg1

        _build_posof(idx_v, posof_v, B)

        for t in range(rpw // L):
            r_v = ridx_v[pl.ds(t * L, L)]
            p_v = plsc.load_gather(posof_v, [r_v])
            pc_v = jnp.maximum(p_v, 0)
            for k in range(L):
                pk, rk, pck = p_v[k], r_v[k], pc_v[k]

                @pl.when(pk >= 0)
                def _(pck=pck, k=k):
                    pltpu.async_copy(
                        val_hbm.at[pl.ds(pck, 1)],
                        rows_v.at[pl.ds(k, 1)], semg,
                    )

                @pl.when(pk < 0)
                def _(rk=rk, k=k):
                    pltpu.async_copy(
                        mem_hbm.at[pl.ds(rk, 1)],
                        rows_v.at[pl.ds(k, 1)], semg,
                    )
            for k in range(L):
                pltpu.make_async_copy(
                    mem_hbm.at[pl.ds(0, 1)], rows_v.at[pl.ds(k, 1)], semg
                ).wait()
            pltpu.sync_copy(rows_v, out_hbm.at[pl.ds(base + t * L, L)])

    return pl.kernel(
        body,
        out_type=jax.ShapeDtypeStruct((R, D), jnp.float32),
        mesh=_mesh(),
        scratch_types=[
            pltpu.VMEM((B,), jnp.int32),
            pltpu.VMEM((M,), jnp.int32),
            pltpu.VMEM((rpw,), jnp.int32),
            pltpu.VMEM((L, D), jnp.float32),
            pltpu.SemaphoreType.DMA,
        ],
        compiler_params=_params(),
        cost_estimate=pl.CostEstimate(
            flops=0, transcendentals=0, bytes_accessed=2 * R * D * 4
        ),
    )


def _make_copy(M, D):
    NCH = 8
    rows = M // NCH

    def body(src, dst, sem):
        descs = [
            pltpu.async_copy(
                src.at[pl.ds(i * rows, rows)],
                dst.at[pl.ds(i * rows, rows)],
                sem,
            )
            for i in range(NCH)
        ]
        for d in descs:
            d.wait()

    return pl.pallas_call(
        body,
        in_specs=[pl.BlockSpec(memory_space=pltpu.MemorySpace.HBM)],
        out_specs=pl.BlockSpec(memory_space=pltpu.MemorySpace.HBM),
        out_shape=jax.ShapeDtypeStruct((M, D), jnp.float32),
        scratch_shapes=[pltpu.SemaphoreType.DMA],
    )


def kernel(mem, val, idx, retrieve_idx):
    M, D = mem.shape
    B = idx.shape[0]
    R = retrieve_idx.shape[0]

    retrieved = _make_retrieve(M, D, B, R)(mem, val, idx, retrieve_idx)
    new_mem_ref = jax.new_ref(_make_copy(M, D)(mem))
    _make_update(M, D, B)(val, idx, new_mem_ref)
    new_mem = jax.freeze(new_mem_ref)
    return new_mem, retrieved


# speedup vs baseline: 33.4257x; 33.4257x over previous
import jax
import jax.numpy as jnp
from jax import lax
from jax.experimental import pallas as pl
from jax.experimental.pallas import tpu as pltpu
from jax.experimental.pallas import tpu_sc as plsc

NC, NS, L = 2, 16, 16
NW = NC * NS


def _mesh():
    return plsc.VectorSubcoreMesh(
        core_axis_name="c", subcore_axis_name="s", num_cores=NC, num_subcores=NS
    )


def _params():
    return pltpu.CompilerParams(needs_layout_passes=False)


def _worker_id():
    return lax.axis_index("s") * NC + lax.axis_index("c")


def _build_posof(idx_v, posof_v, B):
    lid = lax.iota(jnp.int32, L)

    @pl.loop(0, B // L)
    def _(c):
        c_v = idx_v[pl.ds(c * L, L)]
        pos_v = c * L + lid
        for k in range(L):
            plsc.store_scatter(posof_v, [c_v], pos_v, mask=lid == k)


def _make_update(M, D, B):
    bpw = B // NW
    nch = bpw // L

    def body(val_hbm, idx_hbm, new_mem_ref, idx_v, posof_v, sidx, didx,
             rows, gsem, ssem):
        wid = _worker_id()
        base = wid * bpw

        pltpu.sync_copy(idx_hbm, idx_v)
        _build_posof(idx_v, posof_v, B)

        def stage(t):
            c_v = idx_v[pl.ds(base + t * L, L)]
            s_v = plsc.load_gather(posof_v, [c_v])
            p = t % 2
            sidx[p][...] = s_v
            didx[p][...] = c_v
            return pltpu.async_copy(val_hbm.at[sidx[p]], rows[p], gsem[p])

        def scatter(t):
            p = t % 2
            return pltpu.async_copy(
                rows[p], new_mem_ref.at[didx[p]], ssem[p]
            )

        gd = [None, None]
        sd = [None, None]
        gd[0] = stage(0)
        for t in range(nch):
            if t + 1 < nch:
                p2 = (t + 1) % 2
                if sd[p2] is not None:
                    sd[p2].wait()
                    sd[p2] = None
                gd[p2] = stage(t + 1)
            gd[t % 2].wait()
            sd[t % 2] = scatter(t)
        for d in sd:
            if d is not None:
                d.wait()

    return pl.kernel(
        body,
        out_type=(),
        mesh=_mesh(),
        scratch_types=[
            pltpu.VMEM((B,), jnp.int32),
            pltpu.VMEM((M,), jnp.int32),
            [pltpu.VMEM((L,), jnp.int32)] * 2,
            [pltpu.VMEM((L,), jnp.int32)] * 2,
            [pltpu.VMEM((L, D), jnp.float32)] * 2,
            [pltpu.SemaphoreType.DMA] * 2,
            [pltpu.SemaphoreType.DMA] * 2,
        ],
        compiler_params=_params(),
    )


def _make_retrieve(M, D, B, R):
    rpw = R // NW

    def body(mem_hbm, val_hbm, idx_hbm, ridx_hbm, out_hbm, idx_v, posof_v,
             ridx_v, rows_v, semg):
        wid = _worker_id()
        base = wid * rpw

        pltpu.sync_copy(idx_hbm, idx_v)
        pltpu.sync_copy(ridx_hbm.at[pl.ds(base, rpw)], ridx_v)

        neg1 = jnp.full((L,), -1, jnp.int32)

        @pl.loop(0, M // L)
        def _(i):
            posof_v[pl.ds(i * L, L)] = neg1

        _build_posof(idx_v, posof_v, B)

        for t in range(rpw // L):
            r_v = ridx_v[pl.ds(t * L, L)]
            p_v = plsc.load_gather(posof_v, [r_v])
            pc_v = jnp.maximum(p_v, 0)
            for k in range(L):
                pk, rk, pck = p_v[k], r_v[k], pc_v[k]

                @pl.when(pk >= 0)
                def _(pck=pck, k=k):
                    pltpu.async_copy(
                        val_hbm.at[pl.ds(pck, 1)],
                        rows_v.at[pl.ds(k, 1)], semg,
                    )

                @pl.when(pk < 0)
                def _(rk=rk, k=k):
                    pltpu.async_copy(
                        mem_hbm.at[pl.ds(rk, 1)],
                        rows_v.at[pl.ds(k, 1)], semg,
                    )
            for k in range(L):
                pltpu.make_async_copy(
                    mem_hbm.at[pl.ds(0, 1)], rows_v.at[pl.ds(k, 1)], semg
                ).wait()
            pltpu.sync_copy(rows_v, out_hbm.at[pl.ds(base + t * L, L)])

    return pl.kernel(
        body,
        out_type=jax.ShapeDtypeStruct((R, D), jnp.float32),
        mesh=_mesh(),
        scratch_types=[
            pltpu.VMEM((B,), jnp.int32),
            pltpu.VMEM((M,), jnp.int32),
            pltpu.VMEM((rpw,), jnp.int32),
            pltpu.VMEM((L, D), jnp.float32),
            pltpu.SemaphoreType.DMA,
        ],
        compiler_params=_params(),
        cost_estimate=pl.CostEstimate(
            flops=0, transcendentals=0, bytes_accessed=2 * R * D * 4
        ),
    )


def _copy_body(i_ref, o_ref):
    o_ref[...] = i_ref[...]


def _make_copy(M, D):
    blk = 512

    return pl.pallas_call(
        _copy_body,
        grid=(M // blk,),
        in_specs=[pl.BlockSpec((blk, D), lambda i: (i, 0))],
        out_specs=pl.BlockSpec((blk, D), lambda i: (i, 0)),
        out_shape=jax.ShapeDtypeStruct((M, D), jnp.float32),
    )


def kernel(mem, val, idx, retrieve_idx):
    M, D = mem.shape
    B = idx.shape[0]
    R = retrieve_idx.shape[0]

    retrieved = _make_retrieve(M, D, B, R)(mem, val, idx, retrieve_idx)
    new_mem_ref = jax.new_ref(_make_copy(M, D)(mem))
    _make_update(M, D, B)(val, idx, new_mem_ref)
    new_mem = jax.freeze(new_mem_ref)
    return new_mem, retrieved


# speedup vs baseline: 33.8496x; 1.0127x over previous
import jax
import jax.numpy as jnp
from jax import lax
from jax.experimental import pallas as pl
from jax.experimental.pallas import tpu as pltpu
from jax.experimental.pallas import tpu_sc as plsc

NC, NS, L = 2, 16, 16
NW = NC * NS


def _mesh():
    return plsc.VectorSubcoreMesh(
        core_axis_name="c", subcore_axis_name="s", num_cores=NC, num_subcores=NS
    )


def _params():
    return pltpu.CompilerParams(needs_layout_passes=False)


def _worker_id():
    return lax.axis_index("s") * NC + lax.axis_index("c")


def _build_posof(idx_v, posof_v, B):
    lid = lax.iota(jnp.int32, L)

    @pl.loop(0, B // L)
    def _(c):
        c_v = idx_v[pl.ds(c * L, L)]
        pos_v = c * L + lid
        for k in range(L):
            plsc.store_scatter(posof_v, [c_v], pos_v, mask=lid == k)


def _make_update(M, D, B):
    bpw = B // NW
    nch = bpw // L

    def body(val_hbm, idx_hbm, new_mem_ref, idx_v, posof_v, sidx, didx,
             rows, gsem, ssem):
        wid = _worker_id()
        base = wid * bpw

        pltpu.sync_copy(idx_hbm, idx_v)
        _build_posof(idx_v, posof_v, B)

        def stage(t):
            c_v = idx_v[pl.ds(base + t * L, L)]
            s_v = plsc.load_gather(posof_v, [c_v])
            p = t % 2
            sidx[p][...] = s_v
            didx[p][...] = c_v
            return pltpu.async_copy(val_hbm.at[sidx[p]], rows[p], gsem[p])

        def scatter(t):
            p = t % 2
            return pltpu.async_copy(
                rows[p], new_mem_ref.at[didx[p]], ssem[p]
            )

        gd = [None, None]
        sd = [None, None]
        gd[0] = stage(0)
        for t in range(nch):
            if t + 1 < nch:
                p2 = (t + 1) % 2
                if sd[p2] is not None:
                    sd[p2].wait()
                    sd[p2] = None
                gd[p2] = stage(t + 1)
            gd[t % 2].wait()
            sd[t % 2] = scatter(t)
        for d in sd:
            if d is not None:
                d.wait()

    return pl.kernel(
        body,
        out_type=(),
        mesh=_mesh(),
        scratch_types=[
            pltpu.VMEM((B,), jnp.int32),
            pltpu.VMEM((M,), jnp.int32),
            [pltpu.VMEM((L,), jnp.int32)] * 2,
            [pltpu.VMEM((L,), jnp.int32)] * 2,
            [pltpu.VMEM((L, D), jnp.float32)] * 2,
            [pltpu.SemaphoreType.DMA] * 2,
            [pltpu.SemaphoreType.DMA] * 2,
        ],
        compiler_params=_params(),
    )


def _make_retrieve(M, D, B, R):
    rpw = R // NW

    def body(mem_hbm, val_hbm, idx_hbm, ridx_hbm, out_hbm, idx_v, posof_v,
             ridx_v, rows_v, semg):
        wid = _worker_id()
        base = wid * rpw

        pltpu.sync_copy(idx_hbm, idx_v)
        pltpu.sync_copy(ridx_hbm.at[pl.ds(base, rpw)], ridx_v)

        neg1 = jnp.full((L,), -1, jnp.int32)

        @pl.loop(0, M // L)
        def _(i):
            posof_v[pl.ds(i * L, L)] = neg1

        _build_posof(idx_v, posof_v, B)

        for t in range(rpw // L):
            r_v = ridx_v[pl.ds(t * L, L)]
            p_v = plsc.load_gather(posof_v, [r_v])
            pc_v = jnp.maximum(p_v, 0)
            for k in range(L):
                pk, rk, pck = p_v[k], r_v[k], pc_v[k]

                @pl.when(pk >= 0)
                def _(pck=pck, k=k):
                    pltpu.async_copy(
                        val_hbm.at[pl.ds(pck, 1)],
                        rows_v.at[pl.ds(k, 1)], semg,
                    )

                @pl.when(pk < 0)
                def _(rk=rk, k=k):
                    pltpu.async_copy(
                        mem_hbm.at[pl.ds(rk, 1)],
                        rows_v.at[pl.ds(k, 1)], semg,
                    )
            for k in range(L):
                pltpu.make_async_copy(
                    mem_hbm.at[pl.ds(0, 1)], rows_v.at[pl.ds(k, 1)], semg
                ).wait()
            pltpu.sync_copy(rows_v, out_hbm.at[pl.ds(base + t * L, L)])

    return pl.kernel(
        body,
        out_type=jax.ShapeDtypeStruct((R, D), jnp.float32),
        mesh=_mesh(),
        scratch_types=[
            pltpu.VMEM((B,), jnp.int32),
            pltpu.VMEM((M,), jnp.int32),
            pltpu.VMEM((rpw,), jnp.int32),
            pltpu.VMEM((L, D), jnp.float32),
            pltpu.SemaphoreType.DMA,
        ],
        compiler_params=_params(),
        cost_estimate=pl.CostEstimate(
            flops=0, transcendentals=0, bytes_accessed=2 * R * D * 4
        ),
    )


def _copy_body(i_ref, o_ref):
    o_ref[...] = i_ref[...]


def _make_copy(M, D):
    blk = 1024

    return pl.pallas_call(
        _copy_body,
        grid=(M // blk,),
        in_specs=[pl.BlockSpec((blk, D), lambda i: (i, 0))],
        out_specs=pl.BlockSpec((blk, D), lambda i: (i, 0)),
        out_shape=jax.ShapeDtypeStruct((M, D), jnp.float32),
    )


def kernel(mem, val, idx, retrieve_idx):
    M, D = mem.shape
    B = idx.shape[0]
    R = retrieve_idx.shape[0]

    retrieved = _make_retrieve(M, D, B, R)(mem, val, idx, retrieve_idx)
    new_mem_ref = jax.new_ref(_make_copy(M, D)(mem))
    _make_update(M, D, B)(val, idx, new_mem_ref)
    new_mem = jax.freeze(new_mem_ref)
    return new_mem, retrieved


# speedup vs baseline: 33.9716x; 1.0036x over previous
import jax
import jax.numpy as jnp
from jax import lax
from jax.experimental import pallas as pl
from jax.experimental.pallas import tpu as pltpu
from jax.experimental.pallas import tpu_sc as plsc

NC, NS, L = 2, 16, 16
NW = NC * NS


def _mesh():
    return plsc.VectorSubcoreMesh(
        core_axis_name="c", subcore_axis_name="s", num_cores=NC, num_subcores=NS
    )


def _params():
    return pltpu.CompilerParams(needs_layout_passes=False)


def _worker_id():
    return lax.axis_index("s") * NC + lax.axis_index("c")


def _build_posof(idx_v, posof_v, B):
    lid = lax.iota(jnp.int32, L)

    @pl.loop(0, B // L)
    def _(c):
        c_v = idx_v[pl.ds(c * L, L)]
        pos_v = c * L + lid
        for k in range(L):
            plsc.store_scatter(posof_v, [c_v], pos_v, mask=lid == k)


def _make_update(M, D, B):
    bpw = B // NW
    nch = bpw // L

    def body(val_hbm, idx_hbm, new_mem_ref, idx_v, posof_v, sidx, didx,
             rows, gsem, ssem):
        wid = _worker_id()
        base = wid * bpw

        pltpu.sync_copy(idx_hbm, idx_v)
        _build_posof(idx_v, posof_v, B)

        def stage(t):
            c_v = idx_v[pl.ds(base + t * L, L)]
            s_v = plsc.load_gather(posof_v, [c_v])
            p = t % 2
            sidx[p][...] = s_v
            didx[p][...] = c_v
            return pltpu.async_copy(val_hbm.at[sidx[p]], rows[p], gsem[p])

        def scatter(t):
            p = t % 2
            return pltpu.async_copy(
                rows[p], new_mem_ref.at[didx[p]], ssem[p]
            )

        gd = [None, None]
        sd = [None, None]
        gd[0] = stage(0)
        for t in range(nch):
            if t + 1 < nch:
                p2 = (t + 1) % 2
                if sd[p2] is not None:
                    sd[p2].wait()
                    sd[p2] = None
                gd[p2] = stage(t + 1)
            gd[t % 2].wait()
            sd[t % 2] = scatter(t)
        for d in sd:
            if d is not None:
                d.wait()

    return pl.kernel(
        body,
        out_type=(),
        mesh=_mesh(),
        scratch_types=[
            pltpu.VMEM((B,), jnp.int32),
            pltpu.VMEM((M,), jnp.int32),
            [pltpu.VMEM((L,), jnp.int32)] * 2,
            [pltpu.VMEM((L,), jnp.int32)] * 2,
            [pltpu.VMEM((L, D), jnp.float32)] * 2,
            [pltpu.SemaphoreType.DMA] * 2,
            [pltpu.SemaphoreType.DMA] * 2,
        ],
        compiler_params=_params(),
    )


def _make_retrieve(M, D, B, R):
    rpw = R // NW

    def body(mem_hbm, val_hbm, idx_hbm, ridx_hbm, out_hbm, idx_v, posof_v,
             ridx_v, rows_v, semg):
        wid = _worker_id()
        base = wid * rpw

        pltpu.sync_copy(idx_hbm, idx_v)
        pltpu.sync_copy(ridx_hbm.at[pl.ds(base, rpw)], ridx_v)

        neg1 = jnp.full((L,), -1, jnp.int32)

        @pl.loop(0, M // L)
        def _(i):
            posof_v[pl.ds(i * L, L)] = neg1

        _build_posof(idx_v, posof_v, B)

        for t in range(rpw // L):
            r_v = ridx_v[pl.ds(t * L, L)]
            p_v = plsc.load_gather(posof_v, [r_v])
            pc_v = jnp.maximum(p_v, 0)
            for k in range(L):
                pk, rk, pck = p_v[k], r_v[k], pc_v[k]

                @pl.when(pk >= 0)
                def _(pck=pck, k=k):
                    pltpu.async_copy(
                        val_hbm.at[pl.ds(pck, 1)],
                        rows_v.at[pl.ds(k, 1)], semg,
                    )

                @pl.when(pk < 0)
                def _(rk=rk, k=k):
                    pltpu.async_copy(
                        mem_hbm.at[pl.ds(rk, 1)],
                        rows_v.at[pl.ds(k, 1)], semg,
                    )
            for k in range(L):
                pltpu.make_async_copy(
                    mem_hbm.at[pl.ds(0, 1)], rows_v.at[pl.ds(k, 1)], semg
                ).wait()
            pltpu.sync_copy(rows_v, out_hbm.at[pl.ds(base + t * L, L)])

    return pl.kernel(
        body,
        out_type=jax.ShapeDtypeStruct((R, D), jnp.float32),
        mesh=_mesh(),
        scratch_types=[
            pltpu.VMEM((B,), jnp.int32),
            pltpu.VMEM((M,), jnp.int32),
            pltpu.VMEM((rpw,), jnp.int32),
            pltpu.VMEM((L, D), jnp.float32),
            pltpu.SemaphoreType.DMA,
        ],
        compiler_params=_params(),
        cost_estimate=pl.CostEstimate(
            flops=0, transcendentals=0, bytes_accessed=2 * R * D * 4
        ),
    )


def _make_copy(M, D):
    CH = 512
    NBUF = 6
    N = M // CH

    def body(src, dst, *rest):
        bufs = rest[:NBUF]
        isems = rest[NBUF:2 * NBUF]
        osems = rest[2 * NBUF:3 * NBUF]

        def read(i, p):
            return pltpu.async_copy(
                src.at[pl.ds(i * CH, CH)], bufs[p], isems[p]
            )

        def write(i, p):
            return pltpu.async_copy(
                bufs[p], dst.at[pl.ds(i * CH, CH)], osems[p]
            )

        ind = [None] * NBUF
        outd = [None] * NBUF
        for j in range(min(NBUF, N)):
            ind[j] = read(j, j)
        for i in range(N):
            p = i % NBUF
            ind[p].wait()
            outd[p] = write(i, p)
            nxt = i + NBUF
            if nxt < N:
                outd[p].wait()
                outd[p] = None
                ind[p] = read(nxt, p)
        for d in outd:
            if d is not None:
                d.wait()

    return pl.pallas_call(
        body,
        in_specs=[pl.BlockSpec(memory_space=pltpu.MemorySpace.HBM)],
        out_specs=pl.BlockSpec(memory_space=pltpu.MemorySpace.HBM),
        out_shape=jax.ShapeDtypeStruct((M, D), jnp.float32),
        scratch_shapes=(
            [pltpu.VMEM((CH, D), jnp.float32)] * NBUF
            + [pltpu.SemaphoreType.DMA] * (2 * NBUF)
        ),
    )


def kernel(mem, val, idx, retrieve_idx):
    M, D = mem.shape
    B = idx.shape[0]
    R = retrieve_idx.shape[0]

    retrieved = _make_retrieve(M, D, B, R)(mem, val, idx, retrieve_idx)
    new_mem_ref = jax.new_ref(_make_copy(M, D)(mem))
    _make_update(M, D, B)(val, idx, new_mem_ref)
    new_mem = jax.freeze(new_mem_ref)
    return new_mem, retrieved
